# baseline (speedup 1.0000x reference)
import dataclasses
import functools
import math

import jax
import jax.numpy as jnp
from jax import lax
from jax.experimental import pallas as pl
from jax.experimental.pallas import tpu as pltpu
from jax.experimental.pallas import tpu_sc as plsc

N = 1024
V = 100000
PAD = 0
EPS = 0.1 / (V - 1)
CONF = 0.9
C1 = (V - 2) * EPS * math.log(EPS) + CONF * math.log(CONF)

_L = 16
_NW = 32
_RPW = N // _NW

_BC = 4096
_NC = (V + _BC - 1) // _BC


def _sc_gather(x, tgt):
    mesh = plsc.VectorSubcoreMesh(core_axis_name="c", subcore_axis_name="s")
    cp = pltpu.CompilerParams()
    if "needs_layout_passes" in pltpu.CompilerParams.__dataclass_fields__:
        cp = dataclasses.replace(cp, needs_layout_passes=False)

    @functools.partial(
        pl.kernel,
        mesh=mesh,
        compiler_params=cp,
        out_type=(
            jax.ShapeDtypeStruct((N,), jnp.float32),
            jax.ShapeDtypeStruct((N,), jnp.float32),
        ),
        scratch_types=[
            pltpu.VMEM((_RPW,), jnp.int32),
            pltpu.VMEM((_RPW, 8, 128), jnp.float32),
            pltpu.VMEM((_RPW, 128), jnp.float32),
            pltpu.VMEM((_RPW,), jnp.float32),
            pltpu.VMEM((_RPW,), jnp.float32),
            pltpu.SemaphoreType.DMA,
            pltpu.SemaphoreType.DMA,
        ],
    )
    def k(x_hbm, t_hbm, g_hbm, z_hbm, tgt_v, gwin_v, zwin_v, gout_v, zout_v,
          sem_g, sem_z):
        wid = lax.axis_index("s") * 2 + lax.axis_index("c")
        base = wid * _RPW
        pltpu.sync_copy(t_hbm.at[pl.ds(base, _RPW)], tgt_v)
        lane16 = lax.iota(jnp.int32, _L)
        zcp = pltpu.make_async_copy(
            x_hbm.at[pl.ds(base, _RPW), pl.ds(0, 128)], zwin_v, sem_z)
        zcp.start()
        cps = []
        for ch in range(_RPW // _L):
            t16 = tgt_v[pl.ds(ch * _L, _L)]
            for r in range(_L):
                rr = ch * _L + r
                tr = jnp.sum(jnp.where(lane16 == r, t16, 0))
                c0 = pl.multiple_of((tr >> 7) << 7, 128)
                r0 = base + rr - (rr % 8)
                cp = pltpu.make_async_copy(
                    x_hbm.at[pl.ds(r0, 8), pl.ds(c0, 128)],
                    gwin_v.at[rr],
                    sem_g,
                )
                cp.start()
                cps.append(cp)
        for cp in cps:
            cp.wait()
        zcp.wait()
        for ch in range(_RPW // _L):
            t16 = tgt_v[pl.ds(ch * _L, _L)]
            gv = jnp.zeros((_L,), jnp.float32)
            zv = jnp.zeros((_L,), jnp.float32)
            for r in range(_L):
                rr = ch * _L + r
                tr = jnp.sum(jnp.where(lane16 == r, t16, 0))
                lane = tr & 127
                sub = lane >> 4
                l16 = lane & 15
                win = gwin_v[rr, rr % 8, pl.ds(sub * _L, _L)]
                gs = jnp.sum(jnp.where(lane16 == l16, win, 0.0))
                zwin = zwin_v[rr, pl.ds(0, _L)]
                zs = jnp.sum(jnp.where(lane16 == 0, zwin, 0.0))
                gv = jnp.where(lane16 == r, gs, gv)
                zv = jnp.where(lane16 == r, zs, zv)
            gout_v[pl.ds(ch * _L, _L)] = gv
            zout_v[pl.ds(ch * _L, _L)] = zv
        pltpu.sync_copy(gout_v, g_hbm.at[pl.ds(base, _RPW)])
        pltpu.sync_copy(zout_v, z_hbm.at[pl.ds(base, _RPW)])

    return k(x, tgt)


def _tc_body(x_ref, t_ref, g_ref, z_ref, out_ref):
    j = pl.program_id(0)

    @pl.when(j == 0)
    def _init():
        per_row = C1 + EPS * z_ref[...] - (CONF - EPS) * g_ref[...]
        nonpad0 = (t_ref[...] != PAD).astype(jnp.float32)
        out_ref[...] = jnp.sum(per_row * nonpad0).reshape(1, 1)

    tgt = t_ref[...]
    nonpad = (tgt != PAD).astype(jnp.float32)
    col = j * _BC + lax.broadcasted_iota(jnp.int32, (1, _BC), 1)
    xb = jnp.where(col < V, x_ref[...], 0.0)
    rowsum = jnp.sum(xb, axis=1, keepdims=True)
    out_ref[...] += (-EPS * jnp.sum(rowsum * nonpad)).reshape(1, 1)


def kernel(x, target):
    tgt = target.astype(jnp.int32)
    g, z = _sc_gather(x, tgt)
    loss = pl.pallas_call(
        _tc_body,
        grid=(_NC,),
        in_specs=[
            pl.BlockSpec((N, _BC), lambda j: (0, j)),
            pl.BlockSpec((N, 1), lambda j: (0, 0)),
            pl.BlockSpec((N, 1), lambda j: (0, 0)),
            pl.BlockSpec((N, 1), lambda j: (0, 0)),
        ],
        out_specs=pl.BlockSpec((1, 1), lambda j: (0, 0)),
        out_shape=jax.ShapeDtypeStruct((1, 1), jnp.float32),
    )(x, tgt.reshape(N, 1), g.reshape(N, 1), z.reshape(N, 1))
    return jnp.reshape(loss, ())
